# baseline (speedup 1.0000x reference)
import functools

import jax
import jax.numpy as jnp
from jax import lax
from jax.experimental import pallas as pl
from jax.experimental.pallas import tpu as pltpu
from jax.experimental.pallas import tpu_sc as plsc

N = 10000
E = 320000
D = 128

NC = 2
NS = 16
NW = NC * NS
C = 128
G = 16
NG = 5
EPT = NG * G * C
E2 = NW * EPT
NACC = N + 16
RPT = 640
RPT_LAST = N - (NS - 1) * RPT
DW = 128
BN = 1280
GRID = 8


def _mesh():
    return plsc.VectorSubcoreMesh(core_axis_name="c", subcore_axis_name="s",
                                  num_cores=NC, num_subcores=NS)


def _tile_ids():
    c = lax.axis_index("c")
    s = lax.axis_index("s")
    return c, s, s * NC + c


def _rowwise(fn):
    def run(s, r0):
        pl.when(s < NS - 1)(lambda: fn(r0, RPT))
        pl.when(s == NS - 1)(lambda: fn(r0, RPT_LAST))
    return run


@functools.cache
def _make_segsum():
    scratch = [
        pltpu.VMEM((G, C), jnp.int32),
        pltpu.VMEM((G, C), jnp.int32),
        pltpu.VMEM((2, C, D), jnp.float32),
        pltpu.VMEM_SHARED((NACC, D), jnp.float32),
        pltpu.SemaphoreType.DMA,
    ]

    def body(y_hbm, src_hbm, dst_hbm, z_hbm, out_hbm,
             src_v, dst_v, rows_v, acc_sh, sem):
        c, s, wid = _tile_ids()
        r0 = pl.multiple_of(s * RPT, 8)

        def init_rows(rr, nr):
            pltpu.sync_copy(z_hbm.at[pl.ds(0, nr)], acc_sh.at[pl.ds(rr, nr)])

        _rowwise(init_rows)(s, r0)
        plsc.subcore_barrier()

        def group(g, carry):
            pltpu.sync_copy(src_hbm.at[wid, g], src_v)
            pltpu.sync_copy(dst_hbm.at[wid, g], dst_v)

            pltpu.async_copy(y_hbm.at[src_v.at[0]], rows_v.at[0], sem)

            def chunk(j, carry2):
                p = lax.rem(j, 2)
                q = lax.rem(j + 1, 2)

                @pl.when(j < G - 1)
                def _():
                    pltpu.async_copy(y_hbm.at[src_v.at[j + 1]],
                                     rows_v.at[q], sem)

                pltpu.make_async_copy(y_hbm.at[src_v.at[j]],
                                      rows_v.at[p], sem).wait()
                pltpu.sync_copy(rows_v.at[p], acc_sh.at[dst_v.at[j]], add=True)
                return carry2

            return lax.fori_loop(0, G, chunk, carry)

        lax.fori_loop(0, NG, group, 0)
        plsc.subcore_barrier()

        def write_rows(rr, nr):
            pltpu.sync_copy(acc_sh.at[pl.ds(rr, nr)],
                            out_hbm.at[c, pl.ds(rr, nr), :])

        _rowwise(write_rows)(s, r0)

    return pl.kernel(
        body,
        out_type=jax.ShapeDtypeStruct((NC, N, D), jnp.float32),
        mesh=_mesh(),
        scratch_types=scratch,
    )


def _segsum(*args):
    return _make_segsum()(*args)


@functools.cache
def _make_deg():
    scratch = [
        pltpu.VMEM((G, C), jnp.int32),
        pltpu.VMEM((C, DW), jnp.float32),
        pltpu.VMEM_SHARED((NACC, DW), jnp.float32),
    ]

    def body(dst_hbm, z_hbm, on_hbm, out_hbm, dst_v, ones_v, acc_sh):
        c, s, wid = _tile_ids()
        r0 = pl.multiple_of(s * RPT, 8)

        def init_rows(rr, nr):
            pltpu.sync_copy(z_hbm.at[pl.ds(0, nr)], acc_sh.at[pl.ds(rr, nr)])

        _rowwise(init_rows)(s, r0)
        pltpu.sync_copy(on_hbm, ones_v)
        plsc.subcore_barrier()

        def group(g, carry):
            pltpu.sync_copy(dst_hbm.at[wid, g], dst_v)

            def chunk(j, carry2):
                pltpu.sync_copy(ones_v, acc_sh.at[dst_v.at[j]], add=True)
                return carry2

            return lax.fori_loop(0, G, chunk, carry)

        lax.fori_loop(0, NG, group, 0)
        plsc.subcore_barrier()

        def write_rows(rr, nr):
            pltpu.sync_copy(acc_sh.at[pl.ds(rr, nr)],
                            out_hbm.at[c, pl.ds(rr, nr), :])

        _rowwise(write_rows)(s, r0)

    return pl.kernel(
        body,
        out_type=jax.ShapeDtypeStruct((NC, N, DW), jnp.float32),
        mesh=_mesh(),
        scratch_types=scratch,
    )


def _deg(*args):
    return _make_deg()(*args)


def _mm_body(x_ref, w_ref, o_ref):
    o_ref[...] = jnp.dot(x_ref[...], w_ref[...],
                         preferred_element_type=jnp.float32,
                         precision=lax.Precision.HIGHEST)


def _tc_matmul(x, W):
    return pl.pallas_call(
        _mm_body,
        grid=(GRID,),
        in_specs=[pl.BlockSpec((BN, D), lambda i: (i, 0)),
                  pl.BlockSpec((D, D), lambda i: (0, 0))],
        out_specs=pl.BlockSpec((BN, D), lambda i: (i, 0)),
        out_shape=jax.ShapeDtypeStruct((N, D), jnp.float32),
    )(x, W)


def _ep1_body(a_ref, g_ref, y_ref, b_ref, w_ref, o_ref):
    agg = a_ref[0] + a_ref[1] + y_ref[...]
    den = g_ref[0, :, 0:1] + g_ref[1, :, 0:1] + 1.0
    h = agg / den + b_ref[...]
    h = jnp.maximum(h, 0.0)
    o_ref[...] = jnp.dot(h, w_ref[...],
                         preferred_element_type=jnp.float32,
                         precision=lax.Precision.HIGHEST)


def _tc_ep1(a, dp, y, b1, W2):
    return pl.pallas_call(
        _ep1_body,
        grid=(GRID,),
        in_specs=[pl.BlockSpec((2, BN, D), lambda i: (0, i, 0)),
                  pl.BlockSpec((2, BN, DW), lambda i: (0, i, 0)),
                  pl.BlockSpec((BN, D), lambda i: (i, 0)),
                  pl.BlockSpec((1, D), lambda i: (0, 0)),
                  pl.BlockSpec((D, D), lambda i: (0, 0))],
        out_specs=pl.BlockSpec((BN, D), lambda i: (i, 0)),
        out_shape=jax.ShapeDtypeStruct((N, D), jnp.float32),
    )(a, dp, y, b1.reshape(1, D), W2)


def _ep2_body(a_ref, g_ref, y_ref, b_ref, o_ref):
    agg = a_ref[0] + a_ref[1] + y_ref[...]
    den = g_ref[0, :, 0:1] + g_ref[1, :, 0:1] + 1.0
    o_ref[...] = agg / den + b_ref[...]


def _tc_ep2(a, dp, y, b2):
    return pl.pallas_call(
        _ep2_body,
        grid=(GRID,),
        in_specs=[pl.BlockSpec((2, BN, D), lambda i: (0, i, 0)),
                  pl.BlockSpec((2, BN, DW), lambda i: (0, i, 0)),
                  pl.BlockSpec((BN, D), lambda i: (i, 0)),
                  pl.BlockSpec((1, D), lambda i: (0, 0))],
        out_specs=pl.BlockSpec((BN, D), lambda i: (i, 0)),
        out_shape=jax.ShapeDtypeStruct((N, D), jnp.float32),
    )(a, dp, y, b2.reshape(1, D))


def kernel(x, edge_index, W1, b1, W2, b2):
    pad_src = jnp.zeros((E2 - E,), jnp.int32)
    pad_dst = jnp.full((E2 - E,), N, jnp.int32)
    src = jnp.concatenate([edge_index[0], pad_src]).reshape(NW, NG, G, C)
    dst = jnp.concatenate([edge_index[1], pad_dst]).reshape(NW, NG, G, C)
    z = jnp.zeros((RPT, D), jnp.float32)
    on = jnp.ones((C, DW), jnp.float32)

    dp = _deg(dst, z, on)
    W1b, _ = lax.optimization_barrier((W1, dp))
    y1 = _tc_matmul(x, W1b)
    a1 = _segsum(y1, src, dst, z)
    y2 = _tc_ep1(a1, dp, y1, b1, W2)
    a2 = _segsum(y2, src, dst, z)
    return _tc_ep2(a2, dp, y2, b2)


# speedup vs baseline: 3.1312x; 3.1312x over previous
import functools

import jax
import jax.numpy as jnp
from jax import lax
from jax.experimental import pallas as pl
from jax.experimental.pallas import tpu as pltpu
from jax.experimental.pallas import tpu_sc as plsc

N = 10000
E = 320000
D = 128

NC = 2
NS = 16
NW = NC * NS
C = 80
G = 25
NG = 5
EPT = NG * G * C
NACC = N
RPT = 640
RPT_LAST = N - (NS - 1) * RPT
DW = 128
BN = 1280
GRID = 8


def _mesh():
    return plsc.VectorSubcoreMesh(core_axis_name="c", subcore_axis_name="s",
                                  num_cores=NC, num_subcores=NS)


def _tile_ids():
    c = lax.axis_index("c")
    s = lax.axis_index("s")
    return c, s, s * NC + c


def _rowwise(fn):
    def run(s, r0):
        pl.when(s < NS - 1)(lambda: fn(r0, RPT))
        pl.when(s == NS - 1)(lambda: fn(r0, RPT_LAST))
    return run


@functools.cache
def _make_segsum():
    scratch = [
        pltpu.VMEM((G, C), jnp.int32),
        pltpu.VMEM((G, C), jnp.int32),
        pltpu.VMEM((3, C, D), jnp.float32),
        pltpu.VMEM_SHARED((NACC, D), jnp.float32),
        pltpu.SemaphoreType.DMA,
    ]

    def body(y_hbm, src_hbm, dst_hbm, z_hbm, out_hbm,
             src_v, dst_v, rows_v, acc_sh, sem):
        c, s, wid = _tile_ids()
        r0 = pl.multiple_of(s * RPT, 8)

        def init_rows(rr, nr):
            pltpu.sync_copy(z_hbm.at[pl.ds(0, nr)], acc_sh.at[pl.ds(rr, nr)])

        _rowwise(init_rows)(s, r0)
        plsc.subcore_barrier()

        def group(g, carry):
            pltpu.sync_copy(src_hbm.at[wid, g], src_v)
            pltpu.sync_copy(dst_hbm.at[wid, g], dst_v)

            pltpu.async_copy(y_hbm.at[src_v.at[0]], rows_v.at[0], sem)
            pltpu.async_copy(y_hbm.at[src_v.at[1]], rows_v.at[1], sem)

            def chunk(j, carry2):
                p = lax.rem(j, 3)
                q = lax.rem(j + 2, 3)

                @pl.when(j < G - 2)
                def _():
                    pltpu.async_copy(y_hbm.at[src_v.at[j + 2]],
                                     rows_v.at[q], sem)

                pltpu.make_async_copy(y_hbm.at[src_v.at[j]],
                                      rows_v.at[p], sem).wait()
                pltpu.sync_copy(rows_v.at[p], acc_sh.at[dst_v.at[j]], add=True)
                return carry2

            return lax.fori_loop(0, G, chunk, carry)

        lax.fori_loop(0, NG, group, 0)
        plsc.subcore_barrier()

        def write_rows(rr, nr):
            pltpu.sync_copy(acc_sh.at[pl.ds(rr, nr)],
                            out_hbm.at[c, pl.ds(rr, nr), :])

        _rowwise(write_rows)(s, r0)

    return pl.kernel(
        body,
        out_type=jax.ShapeDtypeStruct((NC, N, D), jnp.float32),
        mesh=_mesh(),
        scratch_types=scratch,
    )


def _segsum(*args):
    return _make_segsum()(*args)


@functools.cache
def _make_deg():
    scratch = [
        pltpu.VMEM((G, C), jnp.int32),
        pltpu.VMEM((C, DW), jnp.float32),
        pltpu.VMEM_SHARED((NACC, DW), jnp.float32),
    ]

    def body(dst_hbm, z_hbm, on_hbm, out_hbm, dst_v, ones_v, acc_sh):
        c, s, wid = _tile_ids()
        r0 = pl.multiple_of(s * RPT, 8)

        def init_rows(rr, nr):
            pltpu.sync_copy(z_hbm.at[pl.ds(0, nr)], acc_sh.at[pl.ds(rr, nr)])

        _rowwise(init_rows)(s, r0)
        pltpu.sync_copy(on_hbm, ones_v)
        plsc.subcore_barrier()

        def group(g, carry):
            pltpu.sync_copy(dst_hbm.at[wid, g], dst_v)

            def chunk(j, carry2):
                pltpu.sync_copy(ones_v, acc_sh.at[dst_v.at[j]], add=True)
                return carry2

            return lax.fori_loop(0, G, chunk, carry)

        lax.fori_loop(0, NG, group, 0)
        plsc.subcore_barrier()

        def write_rows(rr, nr):
            pltpu.sync_copy(acc_sh.at[pl.ds(rr, nr)],
                            out_hbm.at[c, pl.ds(rr, nr), :])

        _rowwise(write_rows)(s, r0)

    return pl.kernel(
        body,
        out_type=jax.ShapeDtypeStruct((NC, N, DW), jnp.float32),
        mesh=_mesh(),
        scratch_types=scratch,
    )


def _deg(*args):
    return _make_deg()(*args)


def _mm_body(x_ref, w_ref, o_ref):
    o_ref[...] = jnp.dot(x_ref[...], w_ref[...],
                         preferred_element_type=jnp.float32,
                         precision=lax.Precision.HIGHEST)


def _tc_matmul(x, W):
    return pl.pallas_call(
        _mm_body,
        grid=(GRID,),
        in_specs=[pl.BlockSpec((BN, D), lambda i: (i, 0)),
                  pl.BlockSpec((D, D), lambda i: (0, 0))],
        out_specs=pl.BlockSpec((BN, D), lambda i: (i, 0)),
        out_shape=jax.ShapeDtypeStruct((N, D), jnp.float32),
    )(x, W)


def _ep1_body(a_ref, g_ref, y_ref, b_ref, w_ref, o_ref):
    agg = a_ref[0] + a_ref[1] + y_ref[...]
    den = g_ref[0, :, 0:1] + g_ref[1, :, 0:1] + 1.0
    h = agg / den + b_ref[...]
    h = jnp.maximum(h, 0.0)
    o_ref[...] = jnp.dot(h, w_ref[...],
                         preferred_element_type=jnp.float32,
                         precision=lax.Precision.HIGHEST)


def _tc_ep1(a, dp, y, b1, W2):
    return pl.pallas_call(
        _ep1_body,
        grid=(GRID,),
        in_specs=[pl.BlockSpec((2, BN, D), lambda i: (0, i, 0)),
                  pl.BlockSpec((2, BN, DW), lambda i: (0, i, 0)),
                  pl.BlockSpec((BN, D), lambda i: (i, 0)),
                  pl.BlockSpec((1, D), lambda i: (0, 0)),
                  pl.BlockSpec((D, D), lambda i: (0, 0))],
        out_specs=pl.BlockSpec((BN, D), lambda i: (i, 0)),
        out_shape=jax.ShapeDtypeStruct((N, D), jnp.float32),
    )(a, dp, y, b1.reshape(1, D), W2)


def _ep2_body(a_ref, g_ref, y_ref, b_ref, o_ref):
    agg = a_ref[0] + a_ref[1] + y_ref[...]
    den = g_ref[0, :, 0:1] + g_ref[1, :, 0:1] + 1.0
    o_ref[...] = agg / den + b_ref[...]


def _tc_ep2(a, dp, y, b2):
    return pl.pallas_call(
        _ep2_body,
        grid=(GRID,),
        in_specs=[pl.BlockSpec((2, BN, D), lambda i: (0, i, 0)),
                  pl.BlockSpec((2, BN, DW), lambda i: (0, i, 0)),
                  pl.BlockSpec((BN, D), lambda i: (i, 0)),
                  pl.BlockSpec((1, D), lambda i: (0, 0))],
        out_specs=pl.BlockSpec((BN, D), lambda i: (i, 0)),
        out_shape=jax.ShapeDtypeStruct((N, D), jnp.float32),
    )(a, dp, y, b2.reshape(1, D))


def kernel(x, edge_index, W1, b1, W2, b2):
    src = edge_index[0].reshape(NW, NG, G, C)
    dst = edge_index[1].reshape(NW, NG, G, C)
    z = jnp.zeros((RPT, D), jnp.float32)
    on = jnp.ones((C, DW), jnp.float32)

    dp = _deg(dst, z, on)
    W1b, _ = lax.optimization_barrier((W1, dp))
    y1 = _tc_matmul(x, W1b)
    a1 = _segsum(y1, src, dst, z)
    y2 = _tc_ep1(a1, dp, y1, b1, W2)
    a2 = _segsum(y2, src, dst, z)
    return _tc_ep2(a2, dp, y2, b2)


# speedup vs baseline: 3.1815x; 1.0161x over previous
import functools

import jax
import jax.numpy as jnp
from jax import lax
from jax.experimental import pallas as pl
from jax.experimental.pallas import tpu as pltpu
from jax.experimental.pallas import tpu_sc as plsc

N = 10000
E = 320000
D = 128

NC = 2
NS = 16
NW = NC * NS
C = 80
G = 25
NG = 5
EPT = NG * G * C
NACC = N
RPT = 640
RPT_LAST = N - (NS - 1) * RPT
DW = 128
BN = 1280
GRID = 8


def _mesh():
    return plsc.VectorSubcoreMesh(core_axis_name="c", subcore_axis_name="s",
                                  num_cores=NC, num_subcores=NS)


def _tile_ids():
    c = lax.axis_index("c")
    s = lax.axis_index("s")
    return c, s, s * NC + c


def _rowwise(fn):
    def run(s, r0):
        pl.when(s < NS - 1)(lambda: fn(r0, RPT))
        pl.when(s == NS - 1)(lambda: fn(r0, RPT_LAST))
    return run


@functools.cache
def _make_segsum():
    scratch = [
        pltpu.VMEM((G, C), jnp.int32),
        pltpu.VMEM((G, C), jnp.int32),
        pltpu.VMEM((3, C, D), jnp.float32),
        pltpu.VMEM_SHARED((NACC, D), jnp.float32),
        pltpu.SemaphoreType.DMA,
    ]

    def body(y_hbm, src_hbm, dst_hbm, z_hbm, out_hbm,
             src_v, dst_v, rows_v, acc_sh, sem):
        c, s, wid = _tile_ids()
        r0 = pl.multiple_of(s * RPT, 8)

        def init_rows(rr, nr):
            pltpu.sync_copy(z_hbm.at[pl.ds(0, nr)], acc_sh.at[pl.ds(rr, nr)])

        _rowwise(init_rows)(s, r0)
        plsc.subcore_barrier()

        def group(g, carry):
            pltpu.sync_copy(src_hbm.at[wid, g], src_v)
            pltpu.sync_copy(dst_hbm.at[wid, g], dst_v)

            pltpu.async_copy(y_hbm.at[src_v.at[0]], rows_v.at[0], sem)
            pltpu.async_copy(y_hbm.at[src_v.at[1]], rows_v.at[1], sem)

            def chunk(j, carry2):
                p = lax.rem(j, 3)
                q = lax.rem(j + 2, 3)

                @pl.when(j < G - 2)
                def _():
                    pltpu.async_copy(y_hbm.at[src_v.at[j + 2]],
                                     rows_v.at[q], sem)

                pltpu.make_async_copy(y_hbm.at[src_v.at[j]],
                                      rows_v.at[p], sem).wait()
                pltpu.sync_copy(rows_v.at[p], acc_sh.at[dst_v.at[j]], add=True)
                return carry2

            return lax.fori_loop(0, G, chunk, carry)

        lax.fori_loop(0, NG, group, 0)
        plsc.subcore_barrier()

        def write_rows(rr, nr):
            pltpu.sync_copy(acc_sh.at[pl.ds(rr, nr)],
                            out_hbm.at[c, pl.ds(rr, nr), :])

        _rowwise(write_rows)(s, r0)

    return pl.kernel(
        body,
        out_type=jax.ShapeDtypeStruct((NC, N, D), jnp.float32),
        mesh=_mesh(),
        scratch_types=scratch,
    )


def _segsum(*args):
    return _make_segsum()(*args)


@functools.cache
def _make_deg():
    scratch = [
        pltpu.VMEM((G, C), jnp.int32),
        pltpu.VMEM((C, DW), jnp.float32),
        pltpu.VMEM_SHARED((NACC, DW), jnp.float32),
    ]

    def body(dst_hbm, z_hbm, on_hbm, out_hbm, dst_v, ones_v, acc_sh):
        c, s, wid = _tile_ids()
        r0 = pl.multiple_of(s * RPT, 8)

        def init_rows(rr, nr):
            pltpu.sync_copy(z_hbm.at[pl.ds(0, nr)], acc_sh.at[pl.ds(rr, nr)])

        _rowwise(init_rows)(s, r0)
        pltpu.sync_copy(on_hbm, ones_v)
        plsc.subcore_barrier()

        def group(g, carry):
            pltpu.sync_copy(dst_hbm.at[wid, g], dst_v)

            def chunk(j, carry2):
                pltpu.sync_copy(ones_v, acc_sh.at[dst_v.at[j]], add=True)
                return carry2

            return lax.fori_loop(0, G, chunk, carry)

        lax.fori_loop(0, NG, group, 0)
        plsc.subcore_barrier()

        def write_rows(rr, nr):
            pltpu.sync_copy(acc_sh.at[pl.ds(rr, nr)],
                            out_hbm.at[c, pl.ds(rr, nr), :])

        _rowwise(write_rows)(s, r0)

    return pl.kernel(
        body,
        out_type=jax.ShapeDtypeStruct((NC, N, DW), jnp.float32),
        mesh=_mesh(),
        scratch_types=scratch,
    )


def _deg(*args):
    return _make_deg()(*args)


def _mm_body(x_ref, w_ref, o_ref):
    o_ref[...] = jnp.dot(x_ref[...], w_ref[...],
                         preferred_element_type=jnp.float32,
                         precision=lax.Precision.HIGHEST)


def _tc_matmul(x, W):
    return pl.pallas_call(
        _mm_body,
        grid=(GRID,),
        in_specs=[pl.BlockSpec((BN, D), lambda i: (i, 0)),
                  pl.BlockSpec((D, D), lambda i: (0, 0))],
        out_specs=pl.BlockSpec((BN, D), lambda i: (i, 0)),
        out_shape=jax.ShapeDtypeStruct((N, D), jnp.float32),
    )(x, W)


def _ep1_body(a_ref, g_ref, y_ref, b_ref, w_ref, o_ref):
    agg = a_ref[0] + a_ref[1] + y_ref[...]
    den = g_ref[0] + g_ref[1] + 1.0
    h = agg / den + b_ref[...]
    h = jnp.maximum(h, 0.0)
    o_ref[...] = jnp.dot(h, w_ref[...],
                         preferred_element_type=jnp.float32,
                         precision=lax.Precision.HIGHEST)


def _tc_ep1(a, dp, y, b1, W2):
    return pl.pallas_call(
        _ep1_body,
        grid=(GRID,),
        in_specs=[pl.BlockSpec((2, BN, D), lambda i: (0, i, 0)),
                  pl.BlockSpec((2, BN, 1), lambda i: (0, i, 0)),
                  pl.BlockSpec((BN, D), lambda i: (i, 0)),
                  pl.BlockSpec((1, D), lambda i: (0, 0)),
                  pl.BlockSpec((D, D), lambda i: (0, 0))],
        out_specs=pl.BlockSpec((BN, D), lambda i: (i, 0)),
        out_shape=jax.ShapeDtypeStruct((N, D), jnp.float32),
    )(a, dp, y, b1.reshape(1, D), W2)


def _ep2_body(a_ref, g_ref, y_ref, b_ref, o_ref):
    agg = a_ref[0] + a_ref[1] + y_ref[...]
    den = g_ref[0] + g_ref[1] + 1.0
    o_ref[...] = agg / den + b_ref[...]


def _tc_ep2(a, dp, y, b2):
    return pl.pallas_call(
        _ep2_body,
        grid=(GRID,),
        in_specs=[pl.BlockSpec((2, BN, D), lambda i: (0, i, 0)),
                  pl.BlockSpec((2, BN, 1), lambda i: (0, i, 0)),
                  pl.BlockSpec((BN, D), lambda i: (i, 0)),
                  pl.BlockSpec((1, D), lambda i: (0, 0))],
        out_specs=pl.BlockSpec((BN, D), lambda i: (i, 0)),
        out_shape=jax.ShapeDtypeStruct((N, D), jnp.float32),
    )(a, dp, y, b2.reshape(1, D))


def kernel(x, edge_index, W1, b1, W2, b2):
    src = edge_index[0].reshape(NW, NG, G, C)
    dst = edge_index[1].reshape(NW, NG, G, C)
    z = jnp.zeros((RPT, D), jnp.float32)
    on = jnp.ones((C, DW), jnp.float32)

    dp = _deg(dst, z, on)[:, :, 0:1]
    y1 = _tc_matmul(x, W1)
    y1b, _ = lax.optimization_barrier((y1, dp))
    a1 = _segsum(y1b, src, dst, z)
    y2 = _tc_ep1(a1, dp, y1, b1, W2)
    a2 = _segsum(y2, src, dst, z)
    return _tc_ep2(a2, dp, y2, b2)


# speedup vs baseline: 3.3493x; 1.0528x over previous
import functools

import jax
import jax.numpy as jnp
from jax import lax
from jax.experimental import pallas as pl
from jax.experimental.pallas import tpu as pltpu
from jax.experimental.pallas import tpu_sc as plsc

N = 10000
E = 320000
D = 128

NC = 2
NS = 16
NW = NC * NS
C = 80
G = 25
NG = 5
NCH = NG * G
EPT = NCH * C
NACC = N
RPT = 640
RPT_LAST = N - (NS - 1) * RPT
DW = 128
BN = 1280
GRID = 8


def _mesh():
    return plsc.VectorSubcoreMesh(core_axis_name="c", subcore_axis_name="s",
                                  num_cores=NC, num_subcores=NS)


def _tile_ids():
    c = lax.axis_index("c")
    s = lax.axis_index("s")
    return c, s, s * NC + c


def _rowwise(fn):
    def run(s, r0):
        pl.when(s < NS - 1)(lambda: fn(r0, RPT))
        pl.when(s == NS - 1)(lambda: fn(r0, RPT_LAST))
    return run


@functools.cache
def _make_segsum():
    scratch = [
        pltpu.VMEM((2, G, C), jnp.int32),
        pltpu.VMEM((2, G, C), jnp.int32),
        pltpu.VMEM((3, C, D), jnp.float32),
        pltpu.VMEM_SHARED((NACC, D), jnp.float32),
        pltpu.SemaphoreType.DMA,
        pltpu.SemaphoreType.DMA,
    ]

    def body(y_hbm, src_hbm, dst_hbm, z_hbm, out_hbm,
             src_v, dst_v, rows_v, acc_sh, sem, sem_i):
        c, s, wid = _tile_ids()
        r0 = pl.multiple_of(s * RPT, 8)

        def init_rows(rr, nr):
            pltpu.sync_copy(z_hbm.at[pl.ds(0, nr)], acc_sh.at[pl.ds(rr, nr)])

        _rowwise(init_rows)(s, r0)
        pltpu.sync_copy(src_hbm.at[wid, 0], src_v.at[0])
        pltpu.sync_copy(dst_hbm.at[wid, 0], dst_v.at[0])
        pltpu.async_copy(src_hbm.at[wid, 1], src_v.at[1], sem_i)
        pltpu.async_copy(dst_hbm.at[wid, 1], dst_v.at[1], sem_i)
        plsc.subcore_barrier()
        pltpu.async_copy(y_hbm.at[src_v.at[0, 0]], rows_v.at[0], sem)
        pltpu.async_copy(y_hbm.at[src_v.at[0, 1]], rows_v.at[1], sem)

        def chunk(j, carry):
            g = lax.div(j, G)
            jj = lax.rem(j, G)
            b = lax.rem(g, 2)

            @pl.when((jj == 0) & (g >= 1) & (g < NG - 1))
            def _():
                pltpu.async_copy(src_hbm.at[wid, g + 1], src_v.at[1 - b],
                                 sem_i)
                pltpu.async_copy(dst_hbm.at[wid, g + 1], dst_v.at[1 - b],
                                 sem_i)

            @pl.when((jj == G - 2) & (g < NG - 1))
            def _():
                pltpu.make_async_copy(src_hbm.at[wid, g + 1],
                                      src_v.at[1 - b], sem_i).wait()
                pltpu.make_async_copy(dst_hbm.at[wid, g + 1],
                                      dst_v.at[1 - b], sem_i).wait()

            @pl.when(j + 2 < NCH)
            def _():
                j2 = j + 2
                b2 = lax.rem(lax.div(j2, G), 2)
                pltpu.async_copy(y_hbm.at[src_v.at[b2, lax.rem(j2, G)]],
                                 rows_v.at[lax.rem(j2, 3)], sem)

            p = lax.rem(j, 3)
            pltpu.make_async_copy(y_hbm.at[src_v.at[b, jj]],
                                  rows_v.at[p], sem).wait()
            pltpu.sync_copy(rows_v.at[p], acc_sh.at[dst_v.at[b, jj]],
                            add=True)
            return carry

        lax.fori_loop(0, NCH, chunk, 0)
        plsc.subcore_barrier()

        def write_rows(rr, nr):
            pltpu.sync_copy(acc_sh.at[pl.ds(rr, nr)],
                            out_hbm.at[c, pl.ds(rr, nr), :])

        _rowwise(write_rows)(s, r0)

    return pl.kernel(
        body,
        out_type=jax.ShapeDtypeStruct((NC, N, D), jnp.float32),
        mesh=_mesh(),
        scratch_types=scratch,
    )


def _segsum(*args):
    return _make_segsum()(*args)


@functools.cache
def _make_deg():
    scratch = [
        pltpu.VMEM((G, C), jnp.int32),
        pltpu.VMEM((C, DW), jnp.float32),
        pltpu.VMEM_SHARED((NACC, DW), jnp.float32),
    ]

    def body(dst_hbm, z_hbm, on_hbm, out_hbm, dst_v, ones_v, acc_sh):
        c, s, wid = _tile_ids()
        r0 = pl.multiple_of(s * RPT, 8)

        def init_rows(rr, nr):
            pltpu.sync_copy(z_hbm.at[pl.ds(0, nr)], acc_sh.at[pl.ds(rr, nr)])

        _rowwise(init_rows)(s, r0)
        pltpu.sync_copy(on_hbm, ones_v)
        plsc.subcore_barrier()

        def group(g, carry):
            pltpu.sync_copy(dst_hbm.at[wid, g], dst_v)

            def chunk(j, carry2):
                pltpu.sync_copy(ones_v, acc_sh.at[dst_v.at[j]], add=True)
                return carry2

            return lax.fori_loop(0, G, chunk, carry)

        lax.fori_loop(0, NG, group, 0)
        plsc.subcore_barrier()

        def write_rows(rr, nr):
            pltpu.sync_copy(acc_sh.at[pl.ds(rr, nr)],
                            out_hbm.at[c, pl.ds(rr, nr), :])

        _rowwise(write_rows)(s, r0)

    return pl.kernel(
        body,
        out_type=jax.ShapeDtypeStruct((NC, N, DW), jnp.float32),
        mesh=_mesh(),
        scratch_types=scratch,
    )


def _deg(*args):
    return _make_deg()(*args)


def _mm_body(x_ref, w_ref, o_ref):
    o_ref[...] = jnp.dot(x_ref[...], w_ref[...],
                         preferred_element_type=jnp.float32,
                         precision=lax.Precision.HIGHEST)


def _tc_matmul(x, W):
    return pl.pallas_call(
        _mm_body,
        grid=(GRID,),
        in_specs=[pl.BlockSpec((BN, D), lambda i: (i, 0)),
                  pl.BlockSpec((D, D), lambda i: (0, 0))],
        out_specs=pl.BlockSpec((BN, D), lambda i: (i, 0)),
        out_shape=jax.ShapeDtypeStruct((N, D), jnp.float32),
    )(x, W)


def _ep1_body(a_ref, g_ref, y_ref, b_ref, w_ref, o_ref):
    agg = a_ref[0] + a_ref[1] + y_ref[...]
    den = g_ref[0] + g_ref[1] + 1.0
    h = agg / den + b_ref[...]
    h = jnp.maximum(h, 0.0)
    o_ref[...] = jnp.dot(h, w_ref[...],
                         preferred_element_type=jnp.float32,
                         precision=lax.Precision.HIGHEST)


def _tc_ep1(a, dp, y, b1, W2):
    return pl.pallas_call(
        _ep1_body,
        grid=(GRID,),
        in_specs=[pl.BlockSpec((2, BN, D), lambda i: (0, i, 0)),
                  pl.BlockSpec((2, BN, 1), lambda i: (0, i, 0)),
                  pl.BlockSpec((BN, D), lambda i: (i, 0)),
                  pl.BlockSpec((1, D), lambda i: (0, 0)),
                  pl.BlockSpec((D, D), lambda i: (0, 0))],
        out_specs=pl.BlockSpec((BN, D), lambda i: (i, 0)),
        out_shape=jax.ShapeDtypeStruct((N, D), jnp.float32),
    )(a, dp, y, b1.reshape(1, D), W2)


def _ep2_body(a_ref, g_ref, y_ref, b_ref, o_ref):
    agg = a_ref[0] + a_ref[1] + y_ref[...]
    den = g_ref[0] + g_ref[1] + 1.0
    o_ref[...] = agg / den + b_ref[...]


def _tc_ep2(a, dp, y, b2):
    return pl.pallas_call(
        _ep2_body,
        grid=(GRID,),
        in_specs=[pl.BlockSpec((2, BN, D), lambda i: (0, i, 0)),
                  pl.BlockSpec((2, BN, 1), lambda i: (0, i, 0)),
                  pl.BlockSpec((BN, D), lambda i: (i, 0)),
                  pl.BlockSpec((1, D), lambda i: (0, 0))],
        out_specs=pl.BlockSpec((BN, D), lambda i: (i, 0)),
        out_shape=jax.ShapeDtypeStruct((N, D), jnp.float32),
    )(a, dp, y, b2.reshape(1, D))


def kernel(x, edge_index, W1, b1, W2, b2):
    src = edge_index[0].reshape(NW, NG, G, C)
    dst = edge_index[1].reshape(NW, NG, G, C)
    z = jnp.zeros((RPT, D), jnp.float32)
    on = jnp.ones((C, DW), jnp.float32)

    dp = _deg(dst, z, on)[:, :, 0:1]
    y1 = _tc_matmul(x, W1)
    y1b, _ = lax.optimization_barrier((y1, dp))
    a1 = _segsum(y1b, src, dst, z)
    y2 = _tc_ep1(a1, dp, y1, b1, W2)
    a2 = _segsum(y2, src, dst, z)
    return _tc_ep2(a2, dp, y2, b2)


# speedup vs baseline: 3.3928x; 1.0130x over previous
import functools

import jax
import jax.numpy as jnp
from jax import lax
from jax.experimental import pallas as pl
from jax.experimental.pallas import tpu as pltpu
from jax.experimental.pallas import tpu_sc as plsc

N = 10000
E = 320000
D = 128

NC = 2
NS = 16
NW = NC * NS
C = 80
G = 25
NG = 5
NCH = NG * G
EPT = NCH * C
NACC = N
RPT = 640
RPT_LAST = N - (NS - 1) * RPT
DW = 128
BN = 1280
GRID = 8


def _mesh():
    return plsc.VectorSubcoreMesh(core_axis_name="c", subcore_axis_name="s",
                                  num_cores=NC, num_subcores=NS)


def _tile_ids():
    c = lax.axis_index("c")
    s = lax.axis_index("s")
    return c, s, s * NC + c


def _rowwise(fn):
    def run(s, r0):
        pl.when(s < NS - 1)(lambda: fn(r0, RPT))
        pl.when(s == NS - 1)(lambda: fn(r0, RPT_LAST))
    return run


@functools.cache
def _make_segsum():
    scratch = [
        pltpu.VMEM((2, G, C), jnp.int32),
        pltpu.VMEM((2, G, C), jnp.int32),
        pltpu.VMEM((3, C, D), jnp.float32),
        pltpu.VMEM_SHARED((NACC, D), jnp.float32),
        pltpu.SemaphoreType.DMA,
        pltpu.SemaphoreType.DMA,
    ]

    def body(y_hbm, src_hbm, dst_hbm, z_hbm, out_hbm,
             src_v, dst_v, rows_v, acc_sh, sem, sem_i):
        c, s, wid = _tile_ids()
        r0 = pl.multiple_of(s * RPT, 8)

        def init_rows(rr, nr):
            pltpu.sync_copy(z_hbm.at[pl.ds(0, nr)], acc_sh.at[pl.ds(rr, nr)])

        _rowwise(init_rows)(s, r0)
        pltpu.sync_copy(src_hbm.at[wid, 0], src_v.at[0])
        pltpu.sync_copy(dst_hbm.at[wid, 0], dst_v.at[0])
        pltpu.async_copy(src_hbm.at[wid, 1], src_v.at[1], sem_i)
        pltpu.async_copy(dst_hbm.at[wid, 1], dst_v.at[1], sem_i)
        plsc.subcore_barrier()
        pltpu.async_copy(y_hbm.at[src_v.at[0, 0]], rows_v.at[0], sem)
        pltpu.async_copy(y_hbm.at[src_v.at[0, 1]], rows_v.at[1], sem)

        def chunk(j, carry):
            g = lax.div(j, G)
            jj = lax.rem(j, G)
            b = lax.rem(g, 2)

            @pl.when((jj == 0) & (g >= 1) & (g < NG - 1))
            def _():
                pltpu.async_copy(src_hbm.at[wid, g + 1], src_v.at[1 - b],
                                 sem_i)
                pltpu.async_copy(dst_hbm.at[wid, g + 1], dst_v.at[1 - b],
                                 sem_i)

            @pl.when((jj == G - 2) & (g < NG - 1))
            def _():
                pltpu.make_async_copy(src_hbm.at[wid, g + 1],
                                      src_v.at[1 - b], sem_i).wait()
                pltpu.make_async_copy(dst_hbm.at[wid, g + 1],
                                      dst_v.at[1 - b], sem_i).wait()

            @pl.when(j + 2 < NCH)
            def _():
                j2 = j + 2
                b2 = lax.rem(lax.div(j2, G), 2)
                pltpu.async_copy(y_hbm.at[src_v.at[b2, lax.rem(j2, G)]],
                                 rows_v.at[lax.rem(j2, 3)], sem)

            p = lax.rem(j, 3)
            pltpu.make_async_copy(y_hbm.at[src_v.at[b, jj]],
                                  rows_v.at[p], sem).wait()
            pltpu.sync_copy(rows_v.at[p], acc_sh.at[dst_v.at[b, jj]],
                            add=True)
            return carry

        lax.fori_loop(0, NCH, chunk, 0)
        plsc.subcore_barrier()

        def write_rows(rr, nr):
            pltpu.sync_copy(acc_sh.at[pl.ds(rr, nr)],
                            out_hbm.at[c, pl.ds(rr, nr), :])

        _rowwise(write_rows)(s, r0)

    return pl.kernel(
        body,
        out_type=jax.ShapeDtypeStruct((NC, N, D), jnp.float32),
        mesh=_mesh(),
        scratch_types=scratch,
    )


def _segsum(*args):
    return _make_segsum()(*args)


@functools.cache
def _make_deg():
    SK = 8

    scratch = [
        pltpu.VMEM((NCH, C), jnp.int32),
        pltpu.VMEM((C, DW), jnp.float32),
        pltpu.VMEM_SHARED((NACC, DW), jnp.float32),
        pltpu.SemaphoreType.DMA,
    ]

    def body(dst_hbm, z_hbm, on_hbm, out_hbm, dst_v, ones_v, acc_sh, sem_s):
        c, s, wid = _tile_ids()
        r0 = pl.multiple_of(s * RPT, 8)

        def init_rows(rr, nr):
            pltpu.sync_copy(z_hbm.at[pl.ds(0, nr)], acc_sh.at[pl.ds(rr, nr)])

        _rowwise(init_rows)(s, r0)
        pltpu.sync_copy(on_hbm, ones_v)
        pltpu.sync_copy(dst_hbm.at[wid], dst_v)
        plsc.subcore_barrier()

        def chunk(j, carry):
            pltpu.async_copy(ones_v, acc_sh.at[dst_v.at[j]], sem_s, add=True)

            @pl.when(j >= SK)
            def _():
                pltpu.make_async_copy(ones_v, acc_sh.at[dst_v.at[j]],
                                      sem_s).wait()

            return carry

        lax.fori_loop(0, NCH, chunk, 0)

        def drain(i, carry):
            pltpu.make_async_copy(ones_v, acc_sh.at[dst_v.at[0]],
                                  sem_s).wait()
            return carry

        lax.fori_loop(0, SK, drain, 0)
        plsc.subcore_barrier()

        def write_rows(rr, nr):
            pltpu.sync_copy(acc_sh.at[pl.ds(rr, nr)],
                            out_hbm.at[c, pl.ds(rr, nr), :])

        _rowwise(write_rows)(s, r0)

    return pl.kernel(
        body,
        out_type=jax.ShapeDtypeStruct((NC, N, DW), jnp.float32),
        mesh=_mesh(),
        scratch_types=scratch,
    )


def _deg(*args):
    return _make_deg()(*args)


def _mm_body(x_ref, w_ref, o_ref):
    o_ref[...] = jnp.dot(x_ref[...], w_ref[...],
                         preferred_element_type=jnp.float32,
                         precision=lax.Precision.HIGHEST)


def _tc_matmul(x, W):
    return pl.pallas_call(
        _mm_body,
        grid=(GRID,),
        in_specs=[pl.BlockSpec((BN, D), lambda i: (i, 0)),
                  pl.BlockSpec((D, D), lambda i: (0, 0))],
        out_specs=pl.BlockSpec((BN, D), lambda i: (i, 0)),
        out_shape=jax.ShapeDtypeStruct((N, D), jnp.float32),
    )(x, W)


def _ep1_body(a_ref, g_ref, y_ref, b_ref, w_ref, o_ref):
    agg = a_ref[0] + a_ref[1] + y_ref[...]
    den = g_ref[0] + g_ref[1] + 1.0
    h = agg / den + b_ref[...]
    h = jnp.maximum(h, 0.0)
    o_ref[...] = jnp.dot(h, w_ref[...],
                         preferred_element_type=jnp.float32,
                         precision=lax.Precision.HIGHEST)


def _tc_ep1(a, dp, y, b1, W2):
    return pl.pallas_call(
        _ep1_body,
        grid=(GRID,),
        in_specs=[pl.BlockSpec((2, BN, D), lambda i: (0, i, 0)),
                  pl.BlockSpec((2, BN, 1), lambda i: (0, i, 0)),
                  pl.BlockSpec((BN, D), lambda i: (i, 0)),
                  pl.BlockSpec((1, D), lambda i: (0, 0)),
                  pl.BlockSpec((D, D), lambda i: (0, 0))],
        out_specs=pl.BlockSpec((BN, D), lambda i: (i, 0)),
        out_shape=jax.ShapeDtypeStruct((N, D), jnp.float32),
    )(a, dp, y, b1.reshape(1, D), W2)


def _ep2_body(a_ref, g_ref, y_ref, b_ref, o_ref):
    agg = a_ref[0] + a_ref[1] + y_ref[...]
    den = g_ref[0] + g_ref[1] + 1.0
    o_ref[...] = agg / den + b_ref[...]


def _tc_ep2(a, dp, y, b2):
    return pl.pallas_call(
        _ep2_body,
        grid=(GRID,),
        in_specs=[pl.BlockSpec((2, BN, D), lambda i: (0, i, 0)),
                  pl.BlockSpec((2, BN, 1), lambda i: (0, i, 0)),
                  pl.BlockSpec((BN, D), lambda i: (i, 0)),
                  pl.BlockSpec((1, D), lambda i: (0, 0))],
        out_specs=pl.BlockSpec((BN, D), lambda i: (i, 0)),
        out_shape=jax.ShapeDtypeStruct((N, D), jnp.float32),
    )(a, dp, y, b2.reshape(1, D))


def kernel(x, edge_index, W1, b1, W2, b2):
    src = edge_index[0].reshape(NW, NG, G, C)
    dst = edge_index[1].reshape(NW, NG, G, C)
    dstd = edge_index[1].reshape(NW, NCH, C)
    z = jnp.zeros((RPT, D), jnp.float32)
    on = jnp.ones((C, DW), jnp.float32)

    dp = _deg(dstd, z, on)[:, :, 0:1]
    y1 = _tc_matmul(x, W1)
    y1b, _ = lax.optimization_barrier((y1, dp))
    a1 = _segsum(y1b, src, dst, z)
    y2 = _tc_ep1(a1, dp, y1, b1, W2)
    a2 = _segsum(y2, src, dst, z)
    return _tc_ep2(a2, dp, y2, b2)
